# Initial kernel scaffold; baseline (speedup 1.0000x reference)
import jax
import jax.numpy as jnp
from jax.experimental import pallas as pl


def kernel(x, word_emb, ng2_emb, ng3_emb, W1, b1, W2, b2):
    raise NotImplementedError("write your pallas kernel here")


# speedup vs baseline: 15.3914x; 15.3914x over previous
import functools

import jax
import jax.numpy as jnp
from jax import lax
from jax.experimental import pallas as pl
from jax.experimental.pallas import tpu as pltpu
from jax.experimental.pallas import tpu_sc as plsc

N_CORES = 2
N_SUBCORES = 16
NW = N_CORES * N_SUBCORES
B = 4096
L = 200
E = 128
H = 256
BPW = B // NW
NPAIR = BPW // 2
C1 = 104
C2 = L - C1

_mesh = plsc.VectorSubcoreMesh(core_axis_name="c", subcore_axis_name="s")


@functools.partial(
    pl.kernel,
    mesh=_mesh,
    out_type=jax.ShapeDtypeStruct((3 * B * E,), jnp.float32),
    scratch_types=[
        pltpu.VMEM((BPW * L,), jnp.int32),
        pltpu.VMEM((L, E), jnp.float32),
        pltpu.VMEM((L, E), jnp.float32),
        pltpu.VMEM((BPW * E,), jnp.float32),
        pltpu.SemaphoreType.DMA,
        pltpu.SemaphoreType.DMA,
    ],
)
def _pool(xw_hbm, xb_hbm, xt_hbm, w_hbm, g2_hbm, g3_hbm, out_hbm,
          idx_v, rows0_v, rows1_v, outb_v, sem0, sem1):
    cid = lax.axis_index("c")
    sid = lax.axis_index("s")
    wid = sid * N_CORES + cid
    base = wid * BPW

    def issue(tbl, i, rows_v, sem):
        pltpu.make_async_copy(tbl.at[idx_v.at[pl.ds(i * L, C1)]],
                              rows_v.at[pl.ds(0, C1)], sem).start()
        pltpu.make_async_copy(tbl.at[idx_v.at[pl.ds(i * L + C1, C2)]],
                              rows_v.at[pl.ds(C1, C2)], sem).start()

    def wait(tbl, rows_v, sem):
        pltpu.make_async_copy(tbl.at[idx_v.at[pl.ds(0, C1)]],
                              rows_v.at[pl.ds(0, C1)], sem).wait()
        pltpu.make_async_copy(tbl.at[idx_v.at[pl.ds(0, C2)]],
                              rows_v.at[pl.ds(C1, C2)], sem).wait()

    def acc(rows_v, i):
        def row_body(r, accs):
            return tuple(a + rows_v[r, pl.ds(16 * j, 16)]
                         for j, a in enumerate(accs))

        zeros = tuple(jnp.zeros((16,), jnp.float32) for _ in range(8))
        accs = lax.fori_loop(0, L, row_body, zeros)
        for j, a in enumerate(accs):
            outb_v[pl.ds(i * E + 16 * j, 16)] = a

    for t, tbl, xids in ((0, w_hbm, xw_hbm), (1, g2_hbm, xb_hbm),
                         (2, g3_hbm, xt_hbm)):
        pltpu.sync_copy(xids.at[pl.ds(base * L, BPW * L)], idx_v)
        issue(tbl, 0, rows0_v, sem0)

        def pair_body(p, carry, tbl=tbl):
            e = 2 * p
            o = 2 * p + 1
            nxt = jnp.where(e + 2 >= BPW, 0, e + 2)
            issue(tbl, o, rows1_v, sem1)
            wait(tbl, rows0_v, sem0)
            acc(rows0_v, e)
            issue(tbl, nxt, rows0_v, sem0)
            wait(tbl, rows1_v, sem1)
            acc(rows1_v, o)
            return carry

        lax.fori_loop(0, NPAIR, pair_body, 0)
        wait(tbl, rows0_v, sem0)
        pltpu.sync_copy(outb_v,
                        out_hbm.at[pl.ds((t * B + base) * E, BPW * E)])


def _mlp_body(p_ref, w1_ref, b1_ref, w2_ref, b2_ref, o_ref):
    s = jnp.float32(1.0 / L)
    h = (jnp.dot(p_ref[0] * s, w1_ref[0:E], preferred_element_type=jnp.float32)
         + jnp.dot(p_ref[1] * s, w1_ref[E:2 * E],
                   preferred_element_type=jnp.float32)
         + jnp.dot(p_ref[2] * s, w1_ref[2 * E:3 * E],
                   preferred_element_type=jnp.float32)
         + b1_ref[...])
    h = jnp.maximum(h, 0.0)
    o_ref[...] = (jnp.dot(h, w2_ref[...], preferred_element_type=jnp.float32)
                  + b2_ref[...])


_BLK = 512


def _mlp(pooled, W1, b1r, W2p, b2r):
    return pl.pallas_call(
        _mlp_body,
        grid=(B // _BLK,),
        in_specs=[
            pl.BlockSpec((3, _BLK, E), lambda i: (0, i, 0)),
            pl.BlockSpec((3 * E, H), lambda i: (0, 0)),
            pl.BlockSpec((1, H), lambda i: (0, 0)),
            pl.BlockSpec((H, 128), lambda i: (0, 0)),
            pl.BlockSpec((1, 128), lambda i: (0, 0)),
        ],
        out_specs=pl.BlockSpec((_BLK, 128), lambda i: (i, 0)),
        out_shape=jax.ShapeDtypeStruct((B, 128), jnp.float32),
    )(pooled, W1, b1r, W2p, b2r)


def kernel(x, word_emb, ng2_emb, ng3_emb, W1, b1, W2, b2):
    xw = x[0].reshape(-1)
    xb = x[2].reshape(-1)
    xt = x[3].reshape(-1)
    pooled = _pool(xw, xb, xt, word_emb, ng2_emb, ng3_emb).reshape(3, B, E)
    W2p = jnp.pad(W2, ((0, 0), (0, 128 - W2.shape[1])))
    b2r = jnp.pad(b2, (0, 128 - b2.shape[0])).reshape(1, 128)
    out = _mlp(pooled, W1, b1.reshape(1, H), W2p, b2r)
    return out[:, :10]
